# Initial kernel scaffold; baseline (speedup 1.0000x reference)
import jax
import jax.numpy as jnp
from jax.experimental import pallas as pl


def kernel(x, edge_index, edge_attr, params):
    raise NotImplementedError("write your pallas kernel here")


# speedup vs baseline: 5.4987x; 5.4987x over previous
import functools

import jax
import jax.numpy as jnp
from jax import lax
from jax.experimental import pallas as pl
from jax.experimental.pallas import tpu as pltpu
from jax.experimental.pallas import tpu_sc as plsc

N = 10000
E = 320000
E2 = E + N
C_H = 128
NEG_SLOPE = 0.2

NC = 2
NS = 16
NW = NC * NS
L = 16

EPW = 10368
E2P = EPW * NW
CH = 128
NCHUNK = EPW // CH
NP = 10240
NPS = NP // NS
DUMMY = N

f32 = jnp.float32
i32 = jnp.int32


def _mesh():
    return plsc.VectorSubcoreMesh(
        core_axis_name="c", subcore_axis_name="s", num_cores=NC,
        num_subcores=NS)


def _wid():
    c = lax.axis_index("c")
    s = lax.axis_index("s")
    return s * NC + c, c, s


def _deg_body(dst_hbm, pdeg0, pdeg1, dst2d, ones_buf, zbuf, deg_sh):
    wid, c, s = _wid()
    base = wid * EPW

    @pl.loop(0, NPS // L)
    def _zero(i):
        zbuf[pl.ds(i * L, L)] = jnp.zeros((L,), f32)

    @pl.loop(0, CH // L)
    def _ones(i):
        ones_buf[pl.ds(i * L, L)] = jnp.ones((L,), f32)

    pltpu.sync_copy(zbuf, deg_sh.at[pl.ds(s * NPS, NPS)])
    plsc.subcore_barrier()

    @pl.loop(0, NCHUNK)
    def _chunk(cc):
        pltpu.sync_copy(dst_hbm.at[pl.ds(base + cc * CH, CH)], dst2d.at[cc])
        pltpu.sync_copy(ones_buf, deg_sh.at[dst2d.at[cc]], add=True)

    plsc.subcore_barrier()

    @pl.when(c == 0)
    def _():
        pltpu.sync_copy(deg_sh.at[pl.ds(s * NPS, NPS)],
                        pdeg0.at[pl.ds(s * NPS, NPS)])

    @pl.when(c == 1)
    def _():
        pltpu.sync_copy(deg_sh.at[pl.ds(s * NPS, NPS)],
                        pdeg1.at[pl.ds(s * NPS, NPS)])


def _deg_kernel(dst):
    return pl.kernel(
        _deg_body,
        out_type=[jax.ShapeDtypeStruct((NP,), f32),
                  jax.ShapeDtypeStruct((NP,), f32)],
        mesh=_mesh(),
        scratch_types=[
            pltpu.VMEM((NCHUNK, CH), i32),
            pltpu.VMEM((CH,), f32),
            pltpu.VMEM((NPS,), f32),
            pltpu.VMEM_SHARED((NP,), f32),
        ],
    )(dst)


def _logit_body(xl_hbm, xr_hbm, ee_hbm, src_hbm, dst_hbm, att_hbm,
                logit_hbm, psum0, psum1,
                src_flat, dst2d, att_loc, xl_buf, xr_buf, ee_buf,
                logit_vmem, zbuf, sums_sh):
    wid, c, s = _wid()
    base = wid * EPW

    @pl.loop(0, NPS // L)
    def _zero(i):
        zbuf[pl.ds(i * L, L)] = jnp.zeros((L,), f32)

    pltpu.sync_copy(zbuf, sums_sh.at[pl.ds(s * NPS, NPS)])
    pltpu.sync_copy(src_hbm.at[pl.ds(base, EPW)], src_flat)
    pltpu.sync_copy(att_hbm, att_loc)
    plsc.subcore_barrier()

    lanes = lax.iota(i32, L)
    attv = [att_loc[pl.ds(j * L, L)] for j in range(C_H // L)]

    @pl.loop(0, NCHUNK)
    def _chunk(cc):
        pltpu.sync_copy(dst_hbm.at[pl.ds(base + cc * CH, CH)], dst2d.at[cc])
        pltpu.sync_copy(xl_hbm.at[src_flat.at[pl.ds(cc * CH, CH)]], xl_buf)
        pltpu.sync_copy(xr_hbm.at[dst2d.at[cc]], xr_buf)
        pltpu.sync_copy(ee_hbm.at[pl.ds(base + cc * CH, CH), :], ee_buf)

        @pl.loop(0, CH // L)
        def _group(g):
            lvec = jnp.zeros((L,), f32)
            for t in range(L):
                r = g * L + t
                acc = jnp.zeros((L,), f32)
                for j in range(C_H // L):
                    a = xl_buf[r, pl.ds(j * L, L)]
                    b = xr_buf[r, pl.ds(j * L, L)]
                    e = ee_buf[r, pl.ds(j * L, L)]
                    m = a + b + e
                    v = jnp.maximum(m, NEG_SLOPE * m)
                    acc = acc + v * attv[j]
                sc = jnp.sum(acc)
                lvec = lvec + jnp.where(lanes == t, sc, 0.0)
            logit_vmem[pl.ds(cc * CH + g * L, L)] = lvec

        pltpu.sync_copy(logit_vmem.at[pl.ds(cc * CH, CH)],
                        sums_sh.at[dst2d.at[cc]], add=True)

    pltpu.sync_copy(logit_vmem, logit_hbm.at[pl.ds(base, EPW)])
    plsc.subcore_barrier()

    @pl.when(c == 0)
    def _():
        pltpu.sync_copy(sums_sh.at[pl.ds(s * NPS, NPS)],
                        psum0.at[pl.ds(s * NPS, NPS)])

    @pl.when(c == 1)
    def _():
        pltpu.sync_copy(sums_sh.at[pl.ds(s * NPS, NPS)],
                        psum1.at[pl.ds(s * NPS, NPS)])


def _logit_kernel(xl, xr, ee, src, dst, att):
    return pl.kernel(
        _logit_body,
        out_type=[jax.ShapeDtypeStruct((E2P,), f32),
                  jax.ShapeDtypeStruct((NP,), f32),
                  jax.ShapeDtypeStruct((NP,), f32)],
        mesh=_mesh(),
        compiler_params=pltpu.CompilerParams(needs_layout_passes=False),
        scratch_types=[
            pltpu.VMEM((EPW,), i32),
            pltpu.VMEM((NCHUNK, CH), i32),
            pltpu.VMEM((C_H,), f32),
            pltpu.VMEM((CH, C_H), f32),
            pltpu.VMEM((CH, C_H), f32),
            pltpu.VMEM((CH, C_H), f32),
            pltpu.VMEM((EPW,), f32),
            pltpu.VMEM((NPS,), f32),
            pltpu.VMEM_SHARED((NP,), f32),
        ],
    )(xl, xr, ee, src, dst, att)


def _agg_body(xl_hbm, logit_hbm, c_hbm, src_hbm, dst_hbm,
              pden0, pden1, pout0, pout1,
              src_ch, dst2d, c_loc, logit_ch, ex_ch, rows_buf,
              den_sh, out_sh):
    wid, c, s = _wid()
    base = wid * EPW

    @pl.loop(0, CH)
    def _zr(r):
        for j in range(C_H // L):
            rows_buf[r, pl.ds(j * L, L)] = jnp.zeros((L,), f32)

    for j in range(NPS // CH):
        pltpu.sync_copy(rows_buf,
                        out_sh.at[pl.ds(s * NPS + j * CH, CH), :])
    for j in range(NPS // C_H):
        pltpu.sync_copy(rows_buf.at[0],
                        den_sh.at[pl.ds(s * NPS + j * C_H, C_H)])

    pltpu.sync_copy(c_hbm, c_loc)
    plsc.subcore_barrier()

    @pl.loop(0, NCHUNK)
    def _chunk(cc):
        pltpu.sync_copy(dst_hbm.at[pl.ds(base + cc * CH, CH)], dst2d.at[0])
        pltpu.sync_copy(src_hbm.at[pl.ds(base + cc * CH, CH)], src_ch)
        pltpu.sync_copy(logit_hbm.at[pl.ds(base + cc * CH, CH)], logit_ch)

        @pl.loop(0, CH // L)
        def _ex(g):
            dstv = dst2d[0, pl.ds(g * L, L)]
            lg = logit_ch[pl.ds(g * L, L)]
            cv = plsc.load_gather(c_loc, [dstv])
            ex_ch[pl.ds(g * L, L)] = jnp.exp(lg - cv)

        pltpu.sync_copy(ex_ch, den_sh.at[dst2d.at[0]], add=True)

        pltpu.sync_copy(xl_hbm.at[src_ch], rows_buf)

        @pl.loop(0, CH // L)
        def _scale(g):
            exv = ex_ch[pl.ds(g * L, L)]
            for t in range(L):
                r = g * L + t
                sc = exv[t]
                for j in range(C_H // L):
                    rows_buf[r, pl.ds(j * L, L)] = (
                        rows_buf[r, pl.ds(j * L, L)] * sc)

        pltpu.sync_copy(rows_buf, out_sh.at[dst2d.at[0]], add=True)

    plsc.subcore_barrier()

    @pl.when(c == 0)
    def _():
        pltpu.sync_copy(den_sh.at[pl.ds(s * NPS, NPS)],
                        pden0.at[pl.ds(s * NPS, NPS)])
        pltpu.sync_copy(out_sh.at[pl.ds(s * NPS, NPS), :],
                        pout0.at[pl.ds(s * NPS, NPS), :])

    @pl.when(c == 1)
    def _():
        pltpu.sync_copy(den_sh.at[pl.ds(s * NPS, NPS)],
                        pden1.at[pl.ds(s * NPS, NPS)])
        pltpu.sync_copy(out_sh.at[pl.ds(s * NPS, NPS), :],
                        pout1.at[pl.ds(s * NPS, NPS), :])


def _agg_kernel(xl, logit, c_arr, src, dst):
    return pl.kernel(
        _agg_body,
        out_type=[jax.ShapeDtypeStruct((NP,), f32),
                  jax.ShapeDtypeStruct((NP,), f32),
                  jax.ShapeDtypeStruct((NP, C_H), f32),
                  jax.ShapeDtypeStruct((NP, C_H), f32)],
        mesh=_mesh(),
        compiler_params=pltpu.CompilerParams(needs_layout_passes=False),
        scratch_types=[
            pltpu.VMEM((CH,), i32),
            pltpu.VMEM((1, CH), i32),
            pltpu.VMEM((NP,), f32),
            pltpu.VMEM((CH,), f32),
            pltpu.VMEM((CH,), f32),
            pltpu.VMEM((CH, C_H), f32),
            pltpu.VMEM_SHARED((NP,), f32),
            pltpu.VMEM_SHARED((NP, C_H), f32),
        ],
    )(xl, logit, c_arr, src, dst)


def _ew2_body(a_ref, b_ref, w_ref, o_ref, op):
    o_ref[...] = op(a_ref[...], b_ref[...], w_ref[...])


def _elementwise3(a, b, w, op):
    shape2 = (NP // C_H, C_H)
    body = functools.partial(_ew2_body, op=op)
    out = pl.pallas_call(
        body,
        out_shape=jax.ShapeDtypeStruct(shape2, f32),
    )(a.reshape(shape2), b.reshape(shape2), w.reshape(shape2))
    return out.reshape((NP,))


def _mm0_body(x_ref, wl_ref, wr_ref, xl_ref, xr_ref):
    h = x_ref[...]
    xl_ref[...] = jnp.dot(h, wl_ref[...], preferred_element_type=f32)
    xr_ref[...] = jnp.dot(h, wr_ref[...], preferred_element_type=f32)


def _mm0(x_pad, wl, wr):
    R = 512
    grid = (NP // R,)
    kin = x_pad.shape[1]
    return pl.pallas_call(
        _mm0_body,
        grid=grid,
        in_specs=[pl.BlockSpec((R, kin), lambda i: (i, 0)),
                  pl.BlockSpec((kin, C_H), lambda i: (0, 0)),
                  pl.BlockSpec((kin, C_H), lambda i: (0, 0))],
        out_specs=[pl.BlockSpec((R, C_H), lambda i: (i, 0)),
                   pl.BlockSpec((R, C_H), lambda i: (i, 0))],
        out_shape=[jax.ShapeDtypeStruct((NP, C_H), f32),
                   jax.ShapeDtypeStruct((NP, C_H), f32)],
    )(x_pad, wl, wr)


def _comb_body(p0, p1, d0, d1, b, wl, wr, h_ref, xl_ref, xr_ref, *, R):
    i = pl.program_id(0)
    d = d0[...] + d1[...]
    h = (p0[...] + p1[...]) / (d + 1e-16) + b[...]
    h = jnp.maximum(h, 0.0)
    rows = i * R + lax.broadcasted_iota(i32, (R, C_H), 0)
    h = jnp.where(rows < N, h, 0.0)
    h_ref[...] = h
    xl_ref[...] = jnp.dot(h, wl[...], preferred_element_type=f32)
    xr_ref[...] = jnp.dot(h, wr[...], preferred_element_type=f32)


def _comb_mm(p0, p1, d0, d1, b, wl, wr):
    R = 512
    grid = (NP // R,)
    body = functools.partial(_comb_body, R=R)
    return pl.pallas_call(
        body,
        grid=grid,
        in_specs=[pl.BlockSpec((R, C_H), lambda i: (i, 0)),
                  pl.BlockSpec((R, C_H), lambda i: (i, 0)),
                  pl.BlockSpec((R, 1), lambda i: (i, 0)),
                  pl.BlockSpec((R, 1), lambda i: (i, 0)),
                  pl.BlockSpec((1, C_H), lambda i: (0, 0)),
                  pl.BlockSpec((C_H, C_H), lambda i: (0, 0)),
                  pl.BlockSpec((C_H, C_H), lambda i: (0, 0))],
        out_specs=[pl.BlockSpec((R, C_H), lambda i: (i, 0)),
                   pl.BlockSpec((R, C_H), lambda i: (i, 0)),
                   pl.BlockSpec((R, C_H), lambda i: (i, 0))],
        out_shape=[jax.ShapeDtypeStruct((NP, C_H), f32),
                   jax.ShapeDtypeStruct((NP, C_H), f32),
                   jax.ShapeDtypeStruct((NP, C_H), f32)],
    )(p0, p1, d0.reshape(NP, 1), d1.reshape(NP, 1), b.reshape(1, C_H),
      wl, wr)


def _head_body(p0, p1, d0, d1, b, w1, b1, w2, b2, y_ref, *, R):
    i = pl.program_id(0)
    d = d0[...] + d1[...]
    h = (p0[...] + p1[...]) / (d + 1e-16) + b[...]
    h = jnp.maximum(h, 0.0)
    rows = i * R + lax.broadcasted_iota(i32, (R, C_H), 0)
    h = jnp.where(rows < N, h, 0.0)
    t = jnp.maximum(jnp.dot(h, w1[...], preferred_element_type=f32)
                    + b1[...], 0.0)
    y_ref[...] = jnp.dot(t, w2[...], preferred_element_type=f32) + b2[...]


def _head(p0, p1, d0, d1, b, w1, b1, w2p, b2p):
    R = 512
    grid = (NP // R,)
    body = functools.partial(_head_body, R=R)
    return pl.pallas_call(
        body,
        grid=grid,
        in_specs=[pl.BlockSpec((R, C_H), lambda i: (i, 0)),
                  pl.BlockSpec((R, C_H), lambda i: (i, 0)),
                  pl.BlockSpec((R, 1), lambda i: (i, 0)),
                  pl.BlockSpec((R, 1), lambda i: (i, 0)),
                  pl.BlockSpec((1, C_H), lambda i: (0, 0)),
                  pl.BlockSpec((C_H, C_H), lambda i: (0, 0)),
                  pl.BlockSpec((1, C_H), lambda i: (0, 0)),
                  pl.BlockSpec((C_H, 8), lambda i: (0, 0)),
                  pl.BlockSpec((1, 8), lambda i: (0, 0))],
        out_specs=pl.BlockSpec((R, 8), lambda i: (i, 0)),
        out_shape=jax.ShapeDtypeStruct((NP, 8), f32),
    )(p0, p1, d0.reshape(NP, 1), d1.reshape(NP, 1), b.reshape(1, C_H),
      w1, b1.reshape(1, C_H), w2p, b2p)


def _ee_body(ea_ref, we_ref, o_ref):
    o_ref[...] = jnp.dot(ea_ref[...], we_ref[...], preferred_element_type=f32)


def _ee_mm(ea2, wep):
    R = 1024
    grid = (E2P // R,)
    return pl.pallas_call(
        _ee_body,
        grid=grid,
        in_specs=[pl.BlockSpec((R, 8), lambda i: (i, 0)),
                  pl.BlockSpec((8, C_H), lambda i: (0, 0))],
        out_specs=pl.BlockSpec((R, C_H), lambda i: (i, 0)),
        out_shape=jax.ShapeDtypeStruct((E2P, C_H), f32),
    )(ea2, wep)


def kernel(x, edge_index, edge_attr, params):
    loop = jnp.arange(N, dtype=i32)
    pad_e = E2P - E2
    src = jnp.concatenate(
        [edge_index[0].astype(i32), loop,
         jnp.zeros((pad_e,), i32)])
    dst = jnp.concatenate(
        [edge_index[1].astype(i32), loop,
         jnp.full((pad_e,), DUMMY, i32)])

    ea_mean = jnp.mean(edge_attr, axis=0)
    ea2 = jnp.concatenate(
        [edge_attr, jnp.broadcast_to(ea_mean, (N, edge_attr.shape[1])),
         jnp.zeros((pad_e, edge_attr.shape[1]), f32)], axis=0)
    ea2 = jnp.pad(ea2, ((0, 0), (0, 8 - ea2.shape[1])))

    pdeg0, pdeg1 = _deg_kernel(dst)
    deg_inv = _elementwise3(
        pdeg0, pdeg1, jnp.zeros((NP,), f32),
        lambda a, b, w: 1.0 / jnp.maximum(a + b, 1.0))

    x_pad = jnp.pad(x, ((0, NP - N), (0, 32 - x.shape[1])))
    layers = params['layers']
    wl0 = jnp.pad(layers[0]['Wl'], ((0, 32 - x.shape[1]), (0, 0)))
    wr0 = jnp.pad(layers[0]['Wr'], ((0, 32 - x.shape[1]), (0, 0)))
    xl, xr = _mm0(x_pad, wl0, wr0)

    p0 = p1 = d0 = d1 = None
    for li in range(5):
        p = layers[li]
        if li > 0:
            _, xl, xr = _comb_mm(p0, p1, d0, d1, layers[li - 1]['b'],
                                 p['Wl'], p['Wr'])
        wep = jnp.pad(p['We'], ((0, 8 - p['We'].shape[0]), (0, 0)))
        ee = _ee_mm(ea2, wep)
        logit, ps0, ps1 = _logit_kernel(xl, xr, ee, src, dst, p['att'])
        c_arr = _elementwise3(ps0, ps1, deg_inv,
                              lambda a, b, w: (a + b) * w)
        d0, d1, p0, p1 = _agg_kernel(xl, logit, c_arr, src, dst)

    w2p = jnp.pad(params['W_ff2'], ((0, 0), (0, 7)))
    b2p = jnp.pad(params['b_ff2'], (0, 7)).reshape(1, 8)
    y = _head(p0, p1, d0, d1, layers[4]['b'], params['W_ff1'],
              params['b_ff1'], w2p, b2p)
    return y[:N, :1]


# speedup vs baseline: 6.2840x; 1.1428x over previous
import functools

import jax
import jax.numpy as jnp
from jax import lax
from jax.experimental import pallas as pl
from jax.experimental.pallas import tpu as pltpu
from jax.experimental.pallas import tpu_sc as plsc

N = 10000
E = 320000
E2 = E + N
C_H = 128
NEG_SLOPE = 0.2

NC = 2
NS = 16
NW = NC * NS
L = 16

EPW = 10496
E2P = EPW * NW
CH = 128
NCHUNK = EPW // CH
NP = 10240
NPS = NP // NS
DUMMY = N

f32 = jnp.float32
i32 = jnp.int32


def _mesh():
    return plsc.VectorSubcoreMesh(
        core_axis_name="c", subcore_axis_name="s", num_cores=NC,
        num_subcores=NS)


def _wid():
    c = lax.axis_index("c")
    s = lax.axis_index("s")
    return s * NC + c, c, s


def _deg_body(dst_hbm, pdeg0, pdeg1, dst2d, ones_buf, zbuf, deg_sh):
    wid, c, s = _wid()
    base = wid * EPW

    @pl.loop(0, NPS // L)
    def _zero(i):
        zbuf[pl.ds(i * L, L)] = jnp.zeros((L,), f32)

    @pl.loop(0, CH // L)
    def _ones(i):
        ones_buf[pl.ds(i * L, L)] = jnp.ones((L,), f32)

    pltpu.sync_copy(zbuf, deg_sh.at[pl.ds(s * NPS, NPS)])
    plsc.subcore_barrier()

    @pl.loop(0, NCHUNK)
    def _chunk(cc):
        pltpu.sync_copy(dst_hbm.at[pl.ds(base + cc * CH, CH)], dst2d.at[cc])
        pltpu.sync_copy(ones_buf, deg_sh.at[dst2d.at[cc]], add=True)

    plsc.subcore_barrier()

    @pl.when(c == 0)
    def _():
        pltpu.sync_copy(deg_sh.at[pl.ds(s * NPS, NPS)],
                        pdeg0.at[pl.ds(s * NPS, NPS)])

    @pl.when(c == 1)
    def _():
        pltpu.sync_copy(deg_sh.at[pl.ds(s * NPS, NPS)],
                        pdeg1.at[pl.ds(s * NPS, NPS)])


def _deg_kernel(dst):
    return pl.kernel(
        _deg_body,
        out_type=[jax.ShapeDtypeStruct((NP,), f32),
                  jax.ShapeDtypeStruct((NP,), f32)],
        mesh=_mesh(),
        scratch_types=[
            pltpu.VMEM((NCHUNK, CH), i32),
            pltpu.VMEM((CH,), f32),
            pltpu.VMEM((NPS,), f32),
            pltpu.VMEM_SHARED((NP,), f32),
        ],
    )(dst)


def _logit_body(xl_hbm, xr_hbm, ee_hbm, src_hbm, dst_hbm, att_hbm,
                logit_hbm, psum0, psum1,
                src_flat, dst_ch, att_loc, xl_b0, xl_b1, xr_b0, xr_b1,
                ee_b0, ee_b1, logit_vmem, sums_sh,
                sxl0, sxl1, sxr0, sxr1, see0, see1):
    wid, c, s = _wid()
    base = wid * EPW
    xl_b = (xl_b0, xl_b1)
    xr_b = (xr_b0, xr_b1)
    ee_b = (ee_b0, ee_b1)
    sxl = (sxl0, sxl1)
    sxr = (sxr0, sxr1)
    see = (see0, see1)

    @pl.loop(0, NPS // L)
    def _zero(i):
        logit_vmem[pl.ds(i * L, L)] = jnp.zeros((L,), f32)

    pltpu.sync_copy(logit_vmem.at[pl.ds(0, NPS)],
                    sums_sh.at[pl.ds(s * NPS, NPS)])
    pltpu.sync_copy(src_hbm.at[pl.ds(base, EPW)], src_flat)
    pltpu.sync_copy(att_hbm, att_loc)
    plsc.subcore_barrier()

    lanes = lax.iota(i32, L)
    attv = [att_loc[pl.ds(j * L, L)] for j in range(C_H // L)]

    def _issue(cc, b):
        pltpu.sync_copy(dst_hbm.at[pl.ds(base + cc * CH, CH)], dst_ch.at[b])
        pltpu.async_copy(xl_hbm.at[src_flat.at[pl.ds(cc * CH, CH)]],
                         xl_b[b], sxl[b])
        pltpu.async_copy(xr_hbm.at[dst_ch.at[b]], xr_b[b], sxr[b])
        pltpu.async_copy(ee_hbm.at[pl.ds(base + cc * CH, CH), :],
                         ee_b[b], see[b])

    def _wait(cc, b):
        pltpu.make_async_copy(xl_hbm.at[src_flat.at[pl.ds(cc * CH, CH)]],
                              xl_b[b], sxl[b]).wait()
        pltpu.make_async_copy(xr_hbm.at[dst_ch.at[b]], xr_b[b],
                              sxr[b]).wait()
        pltpu.make_async_copy(ee_hbm.at[pl.ds(base + cc * CH, CH), :],
                              ee_b[b], see[b]).wait()

    _issue(0, 0)

    @pl.loop(0, NCHUNK // 2)
    def _pair(p):
        for b in range(2):
            cc = p * 2 + b
            nxt = cc + 1

            @pl.when(nxt < NCHUNK)
            def _():
                _issue(nxt, 1 - b)

            _wait(cc, b)
            xlb, xrb, eeb = xl_b[b], xr_b[b], ee_b[b]

            @pl.loop(0, CH // L)
            def _group(g):
                lvec = jnp.zeros((L,), f32)
                for t in range(L):
                    r = g * L + t
                    acc = jnp.zeros((L,), f32)
                    for j in range(C_H // L):
                        a = xlb[r, pl.ds(j * L, L)]
                        bb = xrb[r, pl.ds(j * L, L)]
                        e = eeb[r, pl.ds(j * L, L)]
                        m = a + bb + e
                        v = jnp.maximum(m, NEG_SLOPE * m)
                        acc = acc + v * attv[j]
                    sc = jnp.sum(acc)
                    lvec = lvec + jnp.where(lanes == t, sc, 0.0)
                logit_vmem[pl.ds(cc * CH + g * L, L)] = lvec

            pltpu.sync_copy(logit_vmem.at[pl.ds(cc * CH, CH)],
                            sums_sh.at[dst_ch.at[b]], add=True)

    pltpu.sync_copy(logit_vmem, logit_hbm.at[pl.ds(base, EPW)])
    plsc.subcore_barrier()

    @pl.when(c == 0)
    def _():
        pltpu.sync_copy(sums_sh.at[pl.ds(s * NPS, NPS)],
                        psum0.at[pl.ds(s * NPS, NPS)])

    @pl.when(c == 1)
    def _():
        pltpu.sync_copy(sums_sh.at[pl.ds(s * NPS, NPS)],
                        psum1.at[pl.ds(s * NPS, NPS)])


def _logit_kernel(xl, xr, ee, src, dst, att):
    return pl.kernel(
        _logit_body,
        out_type=[jax.ShapeDtypeStruct((E2P,), f32),
                  jax.ShapeDtypeStruct((NP,), f32),
                  jax.ShapeDtypeStruct((NP,), f32)],
        mesh=_mesh(),
        compiler_params=pltpu.CompilerParams(needs_layout_passes=False),
        scratch_types=[
            pltpu.VMEM((EPW,), i32),
            pltpu.VMEM((2, CH), i32),
            pltpu.VMEM((C_H,), f32),
            pltpu.VMEM((CH, C_H), f32),
            pltpu.VMEM((CH, C_H), f32),
            pltpu.VMEM((CH, C_H), f32),
            pltpu.VMEM((CH, C_H), f32),
            pltpu.VMEM((CH, C_H), f32),
            pltpu.VMEM((CH, C_H), f32),
            pltpu.VMEM((EPW,), f32),
            pltpu.VMEM_SHARED((NP,), f32),
            pltpu.SemaphoreType.DMA,
            pltpu.SemaphoreType.DMA,
            pltpu.SemaphoreType.DMA,
            pltpu.SemaphoreType.DMA,
            pltpu.SemaphoreType.DMA,
            pltpu.SemaphoreType.DMA,
        ],
    )(xl, xr, ee, src, dst, att)


def _agg_body(xl_hbm, logit_hbm, c_hbm, src_hbm, dst_hbm,
              pden0, pden1, pout0, pout1,
              src_ch, dst_ch, cg, logit_ch, rows_b0, rows_b1,
              c_sh, den_sh, out_sh, sg0, sg1, so0, so1):
    wid, c, s = _wid()
    base = wid * EPW
    rows_b = (rows_b0, rows_b1)
    sg = (sg0, sg1)
    so = (so0, so1)

    @pl.loop(0, CH)
    def _zr(r):
        for j in range(C_H // L):
            rows_b0[r, pl.ds(j * L, L)] = jnp.zeros((L,), f32)

    for j in range(NPS // CH):
        pltpu.sync_copy(rows_b0,
                        out_sh.at[pl.ds(s * NPS + j * CH, CH), :])
    for j in range(NPS // C_H):
        pltpu.sync_copy(rows_b0.at[0],
                        den_sh.at[pl.ds(s * NPS + j * C_H, C_H)])

    @pl.when(s == 0)
    def _():
        pltpu.sync_copy(c_hbm, c_sh)

    plsc.subcore_barrier()

    def _fetch(cc, b):
        pltpu.sync_copy(dst_hbm.at[pl.ds(base + cc * CH, CH)], dst_ch.at[b])
        pltpu.sync_copy(src_hbm.at[pl.ds(base + cc * CH, CH)], src_ch.at[b])
        pltpu.sync_copy(logit_hbm.at[pl.ds(base + cc * CH, CH)],
                        logit_ch.at[b])
        pltpu.async_copy(xl_hbm.at[src_ch.at[b]], rows_b[b], sg[b])
        pltpu.sync_copy(c_sh.at[dst_ch.at[b]], cg.at[b])

    _fetch(0, 0)

    @pl.loop(0, NCHUNK // 2)
    def _pair(p):
        for b in range(2):
            cc = p * 2 + b
            nxt = cc + 1

            @pl.when(nxt < NCHUNK)
            def _():
                @pl.when(cc > 0)
                def _():
                    pltpu.make_async_copy(
                        rows_b[1 - b], out_sh.at[dst_ch.at[1 - b]],
                        so[1 - b]).wait()

                _fetch(nxt, 1 - b)

            pltpu.make_async_copy(xl_hbm.at[src_ch.at[b]], rows_b[b],
                                  sg[b]).wait()
            rbuf = rows_b[b]

            @pl.loop(0, CH // L)
            def _ex(g):
                lg = logit_ch[b, pl.ds(g * L, L)]
                cv = cg[b, pl.ds(g * L, L)]
                logit_ch[b, pl.ds(g * L, L)] = jnp.exp(lg - cv)

            pltpu.sync_copy(logit_ch.at[b], den_sh.at[dst_ch.at[b]],
                            add=True)

            @pl.loop(0, CH // L)
            def _scale(g):
                exv = logit_ch[b, pl.ds(g * L, L)]
                for t in range(L):
                    r = g * L + t
                    sc = exv[t]
                    for j in range(C_H // L):
                        rbuf[r, pl.ds(j * L, L)] = (
                            rbuf[r, pl.ds(j * L, L)] * sc)

            pltpu.async_copy(rbuf, out_sh.at[dst_ch.at[b]], so[b],
                             add=True)

    for b in range(2):
        pltpu.make_async_copy(rows_b[b], out_sh.at[dst_ch.at[b]],
                              so[b]).wait()

    plsc.subcore_barrier()

    @pl.when(c == 0)
    def _():
        pltpu.sync_copy(den_sh.at[pl.ds(s * NPS, NPS)],
                        pden0.at[pl.ds(s * NPS, NPS)])
        pltpu.sync_copy(out_sh.at[pl.ds(s * NPS, NPS), :],
                        pout0.at[pl.ds(s * NPS, NPS), :])

    @pl.when(c == 1)
    def _():
        pltpu.sync_copy(den_sh.at[pl.ds(s * NPS, NPS)],
                        pden1.at[pl.ds(s * NPS, NPS)])
        pltpu.sync_copy(out_sh.at[pl.ds(s * NPS, NPS), :],
                        pout1.at[pl.ds(s * NPS, NPS), :])


def _agg_kernel(xl, logit, c_arr, src, dst):
    return pl.kernel(
        _agg_body,
        out_type=[jax.ShapeDtypeStruct((NP,), f32),
                  jax.ShapeDtypeStruct((NP,), f32),
                  jax.ShapeDtypeStruct((NP, C_H), f32),
                  jax.ShapeDtypeStruct((NP, C_H), f32)],
        mesh=_mesh(),
        compiler_params=pltpu.CompilerParams(needs_layout_passes=False),
        scratch_types=[
            pltpu.VMEM((2, CH), i32),
            pltpu.VMEM((2, CH), i32),
            pltpu.VMEM((2, CH), f32),
            pltpu.VMEM((2, CH), f32),
            pltpu.VMEM((CH, C_H), f32),
            pltpu.VMEM((CH, C_H), f32),
            pltpu.VMEM_SHARED((NP,), f32),
            pltpu.VMEM_SHARED((NP,), f32),
            pltpu.VMEM_SHARED((NP, C_H), f32),
            pltpu.SemaphoreType.DMA,
            pltpu.SemaphoreType.DMA,
            pltpu.SemaphoreType.DMA,
            pltpu.SemaphoreType.DMA,
        ],
    )(xl, logit, c_arr, src, dst)


def _ew2_body(a_ref, b_ref, w_ref, o_ref, op):
    o_ref[...] = op(a_ref[...], b_ref[...], w_ref[...])


def _elementwise3(a, b, w, op):
    shape2 = (NP // C_H, C_H)
    body = functools.partial(_ew2_body, op=op)
    out = pl.pallas_call(
        body,
        out_shape=jax.ShapeDtypeStruct(shape2, f32),
    )(a.reshape(shape2), b.reshape(shape2), w.reshape(shape2))
    return out.reshape((NP,))


def _mm0_body(x_ref, wl_ref, wr_ref, xl_ref, xr_ref):
    h = x_ref[...]
    xl_ref[...] = jnp.dot(h, wl_ref[...], preferred_element_type=f32)
    xr_ref[...] = jnp.dot(h, wr_ref[...], preferred_element_type=f32)


def _mm0(x_pad, wl, wr):
    R = 512
    grid = (NP // R,)
    kin = x_pad.shape[1]
    return pl.pallas_call(
        _mm0_body,
        grid=grid,
        in_specs=[pl.BlockSpec((R, kin), lambda i: (i, 0)),
                  pl.BlockSpec((kin, C_H), lambda i: (0, 0)),
                  pl.BlockSpec((kin, C_H), lambda i: (0, 0))],
        out_specs=[pl.BlockSpec((R, C_H), lambda i: (i, 0)),
                   pl.BlockSpec((R, C_H), lambda i: (i, 0))],
        out_shape=[jax.ShapeDtypeStruct((NP, C_H), f32),
                   jax.ShapeDtypeStruct((NP, C_H), f32)],
    )(x_pad, wl, wr)


def _comb_body(p0, p1, d0, d1, b, wl, wr, h_ref, xl_ref, xr_ref, *, R):
    i = pl.program_id(0)
    d = d0[...] + d1[...]
    h = (p0[...] + p1[...]) / (d + 1e-16) + b[...]
    h = jnp.maximum(h, 0.0)
    rows = i * R + lax.broadcasted_iota(i32, (R, C_H), 0)
    h = jnp.where(rows < N, h, 0.0)
    h_ref[...] = h
    xl_ref[...] = jnp.dot(h, wl[...], preferred_element_type=f32)
    xr_ref[...] = jnp.dot(h, wr[...], preferred_element_type=f32)


def _comb_mm(p0, p1, d0, d1, b, wl, wr):
    R = 512
    grid = (NP // R,)
    body = functools.partial(_comb_body, R=R)
    return pl.pallas_call(
        body,
        grid=grid,
        in_specs=[pl.BlockSpec((R, C_H), lambda i: (i, 0)),
                  pl.BlockSpec((R, C_H), lambda i: (i, 0)),
                  pl.BlockSpec((R, 1), lambda i: (i, 0)),
                  pl.BlockSpec((R, 1), lambda i: (i, 0)),
                  pl.BlockSpec((1, C_H), lambda i: (0, 0)),
                  pl.BlockSpec((C_H, C_H), lambda i: (0, 0)),
                  pl.BlockSpec((C_H, C_H), lambda i: (0, 0))],
        out_specs=[pl.BlockSpec((R, C_H), lambda i: (i, 0)),
                   pl.BlockSpec((R, C_H), lambda i: (i, 0)),
                   pl.BlockSpec((R, C_H), lambda i: (i, 0))],
        out_shape=[jax.ShapeDtypeStruct((NP, C_H), f32),
                   jax.ShapeDtypeStruct((NP, C_H), f32),
                   jax.ShapeDtypeStruct((NP, C_H), f32)],
    )(p0, p1, d0.reshape(NP, 1), d1.reshape(NP, 1), b.reshape(1, C_H),
      wl, wr)


def _head_body(p0, p1, d0, d1, b, w1, b1, w2, b2, y_ref, *, R):
    i = pl.program_id(0)
    d = d0[...] + d1[...]
    h = (p0[...] + p1[...]) / (d + 1e-16) + b[...]
    h = jnp.maximum(h, 0.0)
    rows = i * R + lax.broadcasted_iota(i32, (R, C_H), 0)
    h = jnp.where(rows < N, h, 0.0)
    t = jnp.maximum(jnp.dot(h, w1[...], preferred_element_type=f32)
                    + b1[...], 0.0)
    y_ref[...] = jnp.dot(t, w2[...], preferred_element_type=f32) + b2[...]


def _head(p0, p1, d0, d1, b, w1, b1, w2p, b2p):
    R = 512
    grid = (NP // R,)
    body = functools.partial(_head_body, R=R)
    return pl.pallas_call(
        body,
        grid=grid,
        in_specs=[pl.BlockSpec((R, C_H), lambda i: (i, 0)),
                  pl.BlockSpec((R, C_H), lambda i: (i, 0)),
                  pl.BlockSpec((R, 1), lambda i: (i, 0)),
                  pl.BlockSpec((R, 1), lambda i: (i, 0)),
                  pl.BlockSpec((1, C_H), lambda i: (0, 0)),
                  pl.BlockSpec((C_H, C_H), lambda i: (0, 0)),
                  pl.BlockSpec((1, C_H), lambda i: (0, 0)),
                  pl.BlockSpec((C_H, 8), lambda i: (0, 0)),
                  pl.BlockSpec((1, 8), lambda i: (0, 0))],
        out_specs=pl.BlockSpec((R, 8), lambda i: (i, 0)),
        out_shape=jax.ShapeDtypeStruct((NP, 8), f32),
    )(p0, p1, d0.reshape(NP, 1), d1.reshape(NP, 1), b.reshape(1, C_H),
      w1, b1.reshape(1, C_H), w2p, b2p)


def _ee_body(ea_ref, we_ref, o_ref):
    o_ref[...] = jnp.dot(ea_ref[...], we_ref[...], preferred_element_type=f32)


def _ee_mm(ea2, wep):
    R = 1024
    grid = (E2P // R,)
    return pl.pallas_call(
        _ee_body,
        grid=grid,
        in_specs=[pl.BlockSpec((R, 8), lambda i: (i, 0)),
                  pl.BlockSpec((8, C_H), lambda i: (0, 0))],
        out_specs=pl.BlockSpec((R, C_H), lambda i: (i, 0)),
        out_shape=jax.ShapeDtypeStruct((E2P, C_H), f32),
    )(ea2, wep)


def kernel(x, edge_index, edge_attr, params):
    loop = jnp.arange(N, dtype=i32)
    pad_e = E2P - E2
    src = jnp.concatenate(
        [edge_index[0].astype(i32), loop,
         jnp.zeros((pad_e,), i32)])
    dst = jnp.concatenate(
        [edge_index[1].astype(i32), loop,
         jnp.full((pad_e,), DUMMY, i32)])

    ea_mean = jnp.mean(edge_attr, axis=0)
    ea2 = jnp.concatenate(
        [edge_attr, jnp.broadcast_to(ea_mean, (N, edge_attr.shape[1])),
         jnp.zeros((pad_e, edge_attr.shape[1]), f32)], axis=0)
    ea2 = jnp.pad(ea2, ((0, 0), (0, 8 - ea2.shape[1])))

    pdeg0, pdeg1 = _deg_kernel(dst)
    deg_inv = _elementwise3(
        pdeg0, pdeg1, jnp.zeros((NP,), f32),
        lambda a, b, w: 1.0 / jnp.maximum(a + b, 1.0))

    x_pad = jnp.pad(x, ((0, NP - N), (0, 32 - x.shape[1])))
    layers = params['layers']
    wl0 = jnp.pad(layers[0]['Wl'], ((0, 32 - x.shape[1]), (0, 0)))
    wr0 = jnp.pad(layers[0]['Wr'], ((0, 32 - x.shape[1]), (0, 0)))
    xl, xr = _mm0(x_pad, wl0, wr0)

    p0 = p1 = d0 = d1 = None
    for li in range(5):
        p = layers[li]
        if li > 0:
            _, xl, xr = _comb_mm(p0, p1, d0, d1, layers[li - 1]['b'],
                                 p['Wl'], p['Wr'])
        wep = jnp.pad(p['We'], ((0, 8 - p['We'].shape[0]), (0, 0)))
        ee = _ee_mm(ea2, wep)
        logit, ps0, ps1 = _logit_kernel(xl, xr, ee, src, dst, p['att'])
        c_arr = _elementwise3(ps0, ps1, deg_inv,
                              lambda a, b, w: (a + b) * w)
        d0, d1, p0, p1 = _agg_kernel(xl, logit, c_arr, src, dst)

    w2p = jnp.pad(params['W_ff2'], ((0, 0), (0, 7)))
    b2p = jnp.pad(params['b_ff2'], (0, 7)).reshape(1, 8)
    y = _head(p0, p1, d0, d1, layers[4]['b'], params['W_ff1'],
              params['b_ff1'], w2p, b2p)
    return y[:N, :1]
